# baseline (speedup 1.0000x reference)
import functools

import jax
import jax.numpy as jnp
from jax import lax
from jax.experimental import pallas as pl
from jax.experimental.pallas import tpu as pltpu
from jax.experimental.pallas import tpu_sc as plsc

_B, _S, _D, _E = 4, 8192, 768, 64
_NW = 32
_SC_CH = 64


def _routing_tail(scores, comb_ref, idx_ref, sc_ref):
    m = jnp.max(scores, axis=-1, keepdims=True)
    ex = jnp.exp(scores - m)
    gates = ex / jnp.sum(ex, axis=-1, keepdims=True)
    iota = jax.lax.broadcasted_iota(jnp.int32, (_B, _E), 1)
    s1 = jnp.max(gates, axis=-1, keepdims=True)
    i1 = jnp.min(jnp.where(gates == s1, iota, _E), axis=-1, keepdims=True)
    masked = jnp.where(iota == i1, -jnp.inf, gates)
    s2 = jnp.max(masked, axis=-1, keepdims=True)
    i2 = jnp.min(jnp.where(masked == s2, iota, _E), axis=-1, keepdims=True)
    denom = s1 + s2 + 1e-9
    comb_ref[...] = (jnp.where(iota == i1, s1 / denom, 0.0)
                     + jnp.where(iota == i2, s2 / denom, 0.0))
    idx_ref[...] = jnp.concatenate([i1, i2], axis=1)
    sc_ref[...] = jnp.concatenate([s1, s2], axis=1)


def _make_sc_partial(rows_per_worker):
    n_ch = rows_per_worker // _SC_CH
    assert rows_per_worker % _SC_CH == 0 and n_ch % 2 == 0
    mesh = plsc.VectorSubcoreMesh(core_axis_name="c", subcore_axis_name="s")

    @functools.partial(
        pl.kernel,
        out_type=jax.ShapeDtypeStruct((_NW, _D), jnp.float32),
        mesh=mesh,
        scratch_types=[
            pltpu.VMEM((_SC_CH, _D), jnp.float32),
            pltpu.VMEM((_SC_CH, _D), jnp.float32),
            pltpu.VMEM((_D,), jnp.float32),
            pltpu.SemaphoreType.DMA,
            pltpu.SemaphoreType.DMA,
        ],
    )
    def sc_partial(x_hbm, out_hbm, buf0, buf1, acc, sem0, sem1):
        wid = lax.axis_index("c") * 16 + lax.axis_index("s")
        base = wid * rows_per_worker
        bufs = (buf0, buf1)
        sems = (sem0, sem1)

        for j in range(_D // 16):
            acc[pl.ds(j * 16, 16)] = jnp.zeros((16,), jnp.float32)

        pltpu.async_copy(x_hbm.at[pl.ds(base, _SC_CH), :], buf0, sem0)
        pltpu.async_copy(x_hbm.at[pl.ds(base + _SC_CH, _SC_CH), :], buf1,
                         sem1)

        def pair_body(g, carry):
            for b in range(2):
                buf, sem = bufs[b], sems[b]
                c = 2 * g + b
                pltpu.make_async_copy(
                    x_hbm.at[pl.ds(base + c * _SC_CH, _SC_CH), :], buf,
                    sem).wait()
                for jg in range(_D // 64):
                    off = jg * 64
                    def row_body(r, acc4, buf=buf, off=off):
                        a0, a1, a2, a3 = acc4
                        for k in range(4):
                            row = r * 4 + k
                            a0 += buf[row, pl.ds(off, 16)]
                            a1 += buf[row, pl.ds(off + 16, 16)]
                            a2 += buf[row, pl.ds(off + 32, 16)]
                            a3 += buf[row, pl.ds(off + 48, 16)]
                        return a0, a1, a2, a3
                    init = (acc[pl.ds(off, 16)], acc[pl.ds(off + 16, 16)],
                            acc[pl.ds(off + 32, 16)],
                            acc[pl.ds(off + 48, 16)])
                    a0, a1, a2, a3 = lax.fori_loop(0, _SC_CH // 4, row_body,
                                                   init)
                    acc[pl.ds(off, 16)] = a0
                    acc[pl.ds(off + 16, 16)] = a1
                    acc[pl.ds(off + 32, 16)] = a2
                    acc[pl.ds(off + 48, 16)] = a3

                @pl.when(c + 2 < n_ch)
                def _(buf=buf, sem=sem, c=c):
                    pltpu.async_copy(
                        x_hbm.at[pl.ds(base + (c + 2) * _SC_CH, _SC_CH), :],
                        buf, sem)
            return carry

        lax.fori_loop(0, n_ch // 2, pair_body, 0)
        pltpu.sync_copy(acc, out_hbm.at[wid])

    return sc_partial


_sc_full_partial = _make_sc_partial((_B * _S) // _NW)


def _finish_body(p_ref, w_ref, b_ref, noise_ref, comb_ref, idx_ref, sc_ref):
    mean = jnp.sum(p_ref[...], axis=1) * (1.0 / _S)
    scores = jnp.dot(mean, w_ref[...], preferred_element_type=jnp.float32)
    scores = scores + b_ref[...][None, :] + noise_ref[...]
    _routing_tail(scores, comb_ref, idx_ref, sc_ref)


@jax.jit
def kernel(x, W_l3, b_l3, noise):
    x2d = x.reshape(_B * _S, _D)
    partials = _sc_full_partial(x2d)
    pgrp = partials.reshape(_B, _NW // _B, _D)
    comb, idx, scores = pl.pallas_call(
        _finish_body,
        out_shape=[
            jax.ShapeDtypeStruct((_B, _E), jnp.float32),
            jax.ShapeDtypeStruct((_B, 2), jnp.int32),
            jax.ShapeDtypeStruct((_B, 2), jnp.float32),
        ],
    )(pgrp, W_l3, b_l3, noise)
    return comb, idx, scores


# speedup vs baseline: 1.2182x; 1.2182x over previous
import functools

import jax
import jax.numpy as jnp
from jax import lax
from jax.experimental import pallas as pl
from jax.experimental.pallas import tpu as pltpu
from jax.experimental.pallas import tpu_sc as plsc

_B, _S, _D, _E = 4, 8192, 768, 64
_NW = 32
_WPB = _NW // _B
_SC_CH = 64

_RPW = 384
_S_TC = _S - _WPB * _RPW
_TC_CHUNK = 512


def _routing_tail(scores, comb_ref, idx_ref, sc_ref):
    m = jnp.max(scores, axis=-1, keepdims=True)
    ex = jnp.exp(scores - m)
    gates = ex / jnp.sum(ex, axis=-1, keepdims=True)
    iota = jax.lax.broadcasted_iota(jnp.int32, (_B, _E), 1)
    s1 = jnp.max(gates, axis=-1, keepdims=True)
    i1 = jnp.min(jnp.where(gates == s1, iota, _E), axis=-1, keepdims=True)
    masked = jnp.where(iota == i1, -jnp.inf, gates)
    s2 = jnp.max(masked, axis=-1, keepdims=True)
    i2 = jnp.min(jnp.where(masked == s2, iota, _E), axis=-1, keepdims=True)
    denom = s1 + s2 + 1e-9
    comb_ref[...] = (jnp.where(iota == i1, s1 / denom, 0.0)
                     + jnp.where(iota == i2, s2 / denom, 0.0))
    idx_ref[...] = jnp.concatenate([i1, i2], axis=1)
    sc_ref[...] = jnp.concatenate([s1, s2], axis=1)


def _make_sc_partial(rows_per_worker, s_tc):
    n_ch = rows_per_worker // _SC_CH
    assert rows_per_worker % _SC_CH == 0 and n_ch % 2 == 0
    mesh = plsc.VectorSubcoreMesh(core_axis_name="c", subcore_axis_name="s")

    @functools.partial(
        pl.kernel,
        out_type=jax.ShapeDtypeStruct((_NW, _D), jnp.float32),
        mesh=mesh,
        scratch_types=[
            pltpu.VMEM((_SC_CH, _D), jnp.float32),
            pltpu.VMEM((_SC_CH, _D), jnp.float32),
            pltpu.VMEM((_D,), jnp.float32),
            pltpu.SemaphoreType.DMA,
            pltpu.SemaphoreType.DMA,
        ],
    )
    def sc_partial(x_hbm, out_hbm, buf0, buf1, acc, sem0, sem1):
        wid = lax.axis_index("c") * 16 + lax.axis_index("s")
        base = (wid // _WPB) * _S + s_tc + (wid % _WPB) * rows_per_worker
        bufs = (buf0, buf1)
        sems = (sem0, sem1)

        for j in range(_D // 16):
            acc[pl.ds(j * 16, 16)] = jnp.zeros((16,), jnp.float32)

        pltpu.async_copy(x_hbm.at[pl.ds(base, _SC_CH), :], buf0, sem0)
        pltpu.async_copy(x_hbm.at[pl.ds(base + _SC_CH, _SC_CH), :], buf1,
                         sem1)

        def pair_body(g, carry):
            for b in range(2):
                buf, sem = bufs[b], sems[b]
                c = 2 * g + b
                pltpu.make_async_copy(
                    x_hbm.at[pl.ds(base + c * _SC_CH, _SC_CH), :], buf,
                    sem).wait()
                for jg in range(_D // 64):
                    off = jg * 64
                    def row_body(r, acc4, buf=buf, off=off):
                        a0, a1, a2, a3 = acc4
                        for k in range(4):
                            row = r * 4 + k
                            a0 += buf[row, pl.ds(off, 16)]
                            a1 += buf[row, pl.ds(off + 16, 16)]
                            a2 += buf[row, pl.ds(off + 32, 16)]
                            a3 += buf[row, pl.ds(off + 48, 16)]
                        return a0, a1, a2, a3
                    init = (acc[pl.ds(off, 16)], acc[pl.ds(off + 16, 16)],
                            acc[pl.ds(off + 32, 16)],
                            acc[pl.ds(off + 48, 16)])
                    a0, a1, a2, a3 = lax.fori_loop(0, _SC_CH // 4, row_body,
                                                   init)
                    acc[pl.ds(off, 16)] = a0
                    acc[pl.ds(off + 16, 16)] = a1
                    acc[pl.ds(off + 32, 16)] = a2
                    acc[pl.ds(off + 48, 16)] = a3

                @pl.when(c + 2 < n_ch)
                def _(buf=buf, sem=sem, c=c):
                    pltpu.async_copy(
                        x_hbm.at[pl.ds(base + (c + 2) * _SC_CH, _SC_CH), :],
                        buf, sem)
            return carry

        lax.fori_loop(0, n_ch // 2, pair_body, 0)
        pltpu.sync_copy(acc, out_hbm.at[wid])

    return sc_partial


_sc_partial = _make_sc_partial(_RPW, _S_TC)


def _tc_reduce_body(x_ref, out_ref):
    r = pl.program_id(0)

    @pl.when(r == 0)
    def _init():
        out_ref[...] = jnp.zeros_like(out_ref)

    out_ref[...] += jnp.sum(x_ref[...], axis=1)


def _finish_body(ptc_ref, psc_ref, w_ref, b_ref, noise_ref, comb_ref,
                 idx_ref, sc_ref):
    mean = (ptc_ref[...] + jnp.sum(psc_ref[...], axis=1)) * (1.0 / _S)
    scores = jnp.dot(mean, w_ref[...], preferred_element_type=jnp.float32)
    scores = scores + b_ref[...][None, :] + noise_ref[...]
    _routing_tail(scores, comb_ref, idx_ref, sc_ref)


@jax.jit
def kernel(x, W_l3, b_l3, noise):
    x2d = x.reshape(_B * _S, _D)
    psc = _sc_partial(x2d)
    ptc = pl.pallas_call(
        _tc_reduce_body,
        grid=(_S_TC // _TC_CHUNK,),
        in_specs=[pl.BlockSpec((_B, _TC_CHUNK, _D), lambda r: (0, r, 0))],
        out_specs=pl.BlockSpec((_B, _D), lambda r: (0, 0)),
        out_shape=jax.ShapeDtypeStruct((_B, _D), jnp.float32),
        compiler_params=pltpu.CompilerParams(
            dimension_semantics=("arbitrary",)),
    )(x)
    comb, idx, scores = pl.pallas_call(
        _finish_body,
        out_shape=[
            jax.ShapeDtypeStruct((_B, _E), jnp.float32),
            jax.ShapeDtypeStruct((_B, 2), jnp.int32),
            jax.ShapeDtypeStruct((_B, 2), jnp.float32),
        ],
    )(ptc, psc.reshape(_B, _WPB, _D), W_l3, b_l3, noise)
    return comb, idx, scores


# speedup vs baseline: 1.5982x; 1.3120x over previous
import jax
import jax.numpy as jnp
from jax.experimental import pallas as pl
from jax.experimental.pallas import tpu as pltpu

_B, _S, _D, _E = 4, 8192, 768, 64
_RCHUNK = 1024


def _routing_tail(scores, comb_ref, idx_ref, sc_ref):
    m = jnp.max(scores, axis=-1, keepdims=True)
    ex = jnp.exp(scores - m)
    gates = ex / jnp.sum(ex, axis=-1, keepdims=True)
    iota = jax.lax.broadcasted_iota(jnp.int32, (_B, _E), 1)
    s1 = jnp.max(gates, axis=-1, keepdims=True)
    i1 = jnp.min(jnp.where(gates == s1, iota, _E), axis=-1, keepdims=True)
    masked = jnp.where(iota == i1, -jnp.inf, gates)
    s2 = jnp.max(masked, axis=-1, keepdims=True)
    i2 = jnp.min(jnp.where(masked == s2, iota, _E), axis=-1, keepdims=True)
    denom = s1 + s2 + 1e-9
    comb_ref[...] = (jnp.where(iota == i1, s1 / denom, 0.0)
                     + jnp.where(iota == i2, s2 / denom, 0.0))
    idx_ref[...] = jnp.concatenate([i1, i2], axis=1)
    sc_ref[...] = jnp.concatenate([s1, s2], axis=1)


def _body(x_ref, w_ref, b_ref, noise_ref, comb_ref, idx_ref, sc_ref, acc_ref):
    r = pl.program_id(0)
    nr = pl.num_programs(0)

    @pl.when(r == 0)
    def _init():
        acc_ref[...] = jnp.zeros_like(acc_ref)

    bi = (r * _RCHUNK) // _S
    acc_ref[pl.ds(bi, 1), :] += jnp.sum(x_ref[...], axis=0)[None, :]

    @pl.when(r == nr - 1)
    def _epilogue():
        mean = acc_ref[...] * (1.0 / _S)
        scores = jnp.dot(mean, w_ref[...], preferred_element_type=jnp.float32)
        scores = scores + b_ref[...][None, :] + noise_ref[...]
        _routing_tail(scores, comb_ref, idx_ref, sc_ref)


@jax.jit
def kernel(x, W_l3, b_l3, noise):
    x2d = x.reshape(_B * _S, _D)
    nr = (_B * _S) // _RCHUNK
    comb, idx, scores = pl.pallas_call(
        _body,
        grid=(nr,),
        in_specs=[
            pl.BlockSpec((_RCHUNK, _D), lambda r: (r, 0)),
            pl.BlockSpec((_D, _E), lambda r: (0, 0)),
            pl.BlockSpec((_E,), lambda r: (0,)),
            pl.BlockSpec((_B, _E), lambda r: (0, 0)),
        ],
        out_specs=[
            pl.BlockSpec((_B, _E), lambda r: (0, 0)),
            pl.BlockSpec((_B, 2), lambda r: (0, 0)),
            pl.BlockSpec((_B, 2), lambda r: (0, 0)),
        ],
        out_shape=[
            jax.ShapeDtypeStruct((_B, _E), jnp.float32),
            jax.ShapeDtypeStruct((_B, 2), jnp.int32),
            jax.ShapeDtypeStruct((_B, 2), jnp.float32),
        ],
        scratch_shapes=[pltpu.VMEM((_B, _D), jnp.float32)],
        compiler_params=pltpu.CompilerParams(
            dimension_semantics=("arbitrary",)),
    )(x2d, W_l3, b_l3, noise)
    return comb, idx, scores
